# baseline (speedup 1.0000x reference)
import functools

import jax
import jax.numpy as jnp
from jax import lax
from jax.experimental import pallas as pl
from jax.experimental.pallas import tpu as pltpu
from jax.experimental.pallas import tpu_sc as plsc

_B = 16384
_L = 50
_D = 64
_VQ = 10000
_VL = 1001
_NC = 2
_NS = 16
_W = _D // _NS
_BSC = _B // _NC
_TCH = 128
_NTCH = _BSC // _TCH
_RPW = _B // (_NC * _NS)
_PCH = 128
_NSL = _D // 16


def _sc_body(tok, lat_i, lon_i, qtabT, t0, ltab, out,
             tab4, tok_d, poolT, shT, outp, latb, lonb, lli, t0v, sh_pool,
             sem_t0, sem_t1, sem_p0, sem_p1, sem_aux, sem_out):
    sid = lax.axis_index("s")
    cid = lax.axis_index("c")
    rbase_sc = cid * _BSC
    cbase = sid * _W

    pltpu.sync_copy(qtabT.at[pl.ds(cbase, _W), :], tab4)
    pltpu.sync_copy(t0, t0v)

    lanes = lax.broadcasted_iota(jnp.int32, (16,), 0)
    zf = jnp.zeros((16,), jnp.float32)
    zi = jnp.zeros((16,), jnp.int32)
    tsems = (sem_t0, sem_t1)
    psems = (sem_p0, sem_p1)

    for b in range(2):
        pltpu.async_copy(tok.at[pl.ds(rbase_sc + b * _TCH, _TCH), :],
                         tok_d.at[b], tsems[b])

    t0c = [plsc.load_gather(t0v, [jnp.full((16,), cbase + c, jnp.int32)])
           for c in range(_W)]

    @pl.loop(0, _NTCH // 2)
    def _pair(pp):
        for b in range(2):
            ch = pp * 2 + b
            pltpu.make_async_copy(tok.at[pl.ds(0, _TCH), :], tok_d.at[b],
                                  tsems[b]).wait()

            @pl.when(pp > 0)
            def _drain():
                for c in range(_W):
                    pltpu.make_async_copy(
                        poolT.at[b, c], sh_pool.at[c, pl.ds(0, _TCH)],
                        psems[b]).wait()

            tok2d = tok_d.at[b]

            @pl.loop(0, _TCH // 32)
            def _grp(d):
                rows = (lanes + d * 32, lanes + d * 32 + 16)

                def jstep(j, carry):
                    accs = carry
                    jsp = jnp.full((16,), j, jnp.int32)
                    out = []
                    for h in range(2):
                        a = accs[h * 5:h * 5 + 5]
                        tokv = plsc.load_gather(tok2d, [rows[h], jsp])
                        cnt = a[4] + jnp.where(
                            tokv == 0, 1, 0).astype(jnp.int32)
                        for c in range(_W):
                            v = plsc.load_gather(
                                tab4, [jnp.full((16,), c, jnp.int32), tokv])
                            out.append(a[c] + v)
                        out.append(cnt)
                    return tuple(out)

                accs = pl.loop(
                    0, _L,
                    init_carry=(zf, zf, zf, zf, zi, zf, zf, zf, zf, zi),
                    unroll=5)(jstep)

                for h in range(2):
                    a = accs[h * 5:h * 5 + 5]
                    nf = a[4].astype(jnp.float32)
                    inv = 1.0 / jnp.maximum(jnp.float32(_L) - nf, 1.0)
                    for c in range(_W):
                        poolT[b, c, pl.ds(d * 32 + h * 16, 16)] = \
                            (a[c] - nf * t0c[c]) * inv

            for c in range(_W):
                pltpu.async_copy(
                    poolT.at[b, c],
                    sh_pool.at[cbase + c, pl.ds(ch * _TCH, _TCH)], psems[b])

            @pl.when(pp < _NTCH // 2 - 1)
            def _pref():
                pltpu.async_copy(
                    tok.at[pl.ds(rbase_sc + (ch + 2) * _TCH, _TCH), :],
                    tok_d.at[b], tsems[b])

    for b in range(2):
        for c in range(_W):
            pltpu.make_async_copy(
                poolT.at[b, c], sh_pool.at[c, pl.ds(0, _TCH)],
                psems[b]).wait()
    plsc.subcore_barrier()

    base0 = rbase_sc + sid * _RPW

    @pl.loop(0, _RPW // _PCH)
    def _piece(p):
        base = base0 + p * _PCH
        lrow = sid * _RPW + p * _PCH

        pltpu.sync_copy(lat_i.at[pl.ds(base, _PCH)], lli.at[0])
        pltpu.sync_copy(lon_i.at[pl.ds(base, _PCH)], lli.at[1])
        cp_lat = pltpu.async_copy(ltab.at[lli.at[0]], latb, sem_aux)
        cp_lon = pltpu.async_copy(ltab.at[lli.at[1]], lonb, sem_aux)

        pltpu.sync_copy(sh_pool.at[:, pl.ds(lrow, _PCH)], shT)

        @pl.loop(0, _PCH)
        def _tr(r):
            rsp = jnp.full((16,), r, jnp.int32)
            for k in range(_NSL):
                outp[r, pl.ds(k * 16, 16)] = plsc.load_gather(
                    shT, [lanes + k * 16, rsp])

        cp_lat.wait()
        cp_lon.wait()
        o1 = pltpu.async_copy(outp, out.at[pl.ds(base, _PCH), pl.ds(0, _D)],
                              sem_out)
        o2 = pltpu.async_copy(latb, out.at[pl.ds(base, _PCH), pl.ds(_D, _D)],
                              sem_out)
        o3 = pltpu.async_copy(lonb,
                              out.at[pl.ds(base, _PCH), pl.ds(2 * _D, _D)],
                              sem_out)
        o1.wait()
        o2.wait()
        o3.wait()


@jax.jit
def _run(tok, lat_i, lon_i, qtabT, t0, ltab):
    mesh = plsc.VectorSubcoreMesh(core_axis_name="c", subcore_axis_name="s")
    return pl.kernel(
        _sc_body,
        out_type=jax.ShapeDtypeStruct((_B, 3 * _D), jnp.float32),
        mesh=mesh,
        scratch_types=[
            pltpu.VMEM((_W, _VQ), jnp.float32),
            pltpu.VMEM((2, _TCH, _L), jnp.int32),
            pltpu.VMEM((2, _W, _TCH), jnp.float32),
            pltpu.VMEM((_D, _PCH), jnp.float32),
            pltpu.VMEM((_PCH, _D), jnp.float32),
            pltpu.VMEM((_PCH, _D), jnp.float32),
            pltpu.VMEM((_PCH, _D), jnp.float32),
            pltpu.VMEM((2, _PCH), jnp.int32),
            pltpu.VMEM((_D,), jnp.float32),
            pltpu.VMEM_SHARED((_D, _BSC), jnp.float32),
            pltpu.SemaphoreType.DMA,
            pltpu.SemaphoreType.DMA,
            pltpu.SemaphoreType.DMA,
            pltpu.SemaphoreType.DMA,
            pltpu.SemaphoreType.DMA,
            pltpu.SemaphoreType.DMA,
        ],
        compiler_params=pltpu.CompilerParams(use_tc_tiling_on_sc=False,
                                             needs_layout_passes=False),
        name="query_model_sc",
    )(tok, lat_i, lon_i, qtabT, t0, ltab)


def kernel(query_tokens, wh_latitude, wh_longitude, query_table, lonlat_table):
    tok = query_tokens.astype(jnp.int32)
    lat_i = wh_latitude.astype(jnp.int32)
    lon_i = wh_longitude.astype(jnp.int32)
    qtab = query_table.astype(jnp.float32)
    return _run(tok, lat_i, lon_i, qtab.T, qtab[0],
                lonlat_table.astype(jnp.float32))


# speedup vs baseline: 1.1835x; 1.1835x over previous
import functools

import jax
import jax.numpy as jnp
from jax import lax
from jax.experimental import pallas as pl
from jax.experimental.pallas import tpu as pltpu
from jax.experimental.pallas import tpu_sc as plsc

_B = 16384
_L = 50
_D = 64
_VQ = 10000
_VL = 1001
_NC = 2
_NS = 16
_W = _D // _NS
_BSC = _B // _NC
_TCH = 128
_NTCH = _BSC // _TCH
_RPW = _B // (_NC * _NS)
_PCH = 128
_NSL = _D // 16


def _sc_body(tok, lat_i, lon_i, qtabT, t0, ltab, out,
             tab4, tok_d, poolT, shT, outp, latb, lonb, lli, t0v, sh_pool,
             sem_t0, sem_t1, sem_p0, sem_p1, sem_aux, sem_out):
    sid = lax.axis_index("s")
    cid = lax.axis_index("c")
    rbase_sc = cid * _BSC
    cbase = sid * _W

    pltpu.sync_copy(qtabT.at[pl.ds(cbase, _W), :], tab4)
    pltpu.sync_copy(t0, t0v)

    lanes = lax.broadcasted_iota(jnp.int32, (16,), 0)
    zf = jnp.zeros((16,), jnp.float32)
    zi = jnp.zeros((16,), jnp.int32)
    tsems = (sem_t0, sem_t1)
    psems = (sem_p0, sem_p1)

    for b in range(2):
        pltpu.async_copy(tok.at[:, pl.ds(rbase_sc + b * _TCH, _TCH)],
                         tok_d.at[b], tsems[b])

    t0c = [plsc.load_gather(t0v, [jnp.full((16,), cbase + c, jnp.int32)])
           for c in range(_W)]

    @pl.loop(0, _NTCH // 2)
    def _pair(pp):
        for b in range(2):
            ch = pp * 2 + b
            pltpu.make_async_copy(tok.at[:, pl.ds(0, _TCH)], tok_d.at[b],
                                  tsems[b]).wait()

            @pl.when(pp > 0)
            def _drain():
                for c in range(_W):
                    pltpu.make_async_copy(
                        poolT.at[b, c], sh_pool.at[c, pl.ds(0, _TCH)],
                        psems[b]).wait()

            @pl.loop(0, _TCH // 16)
            def _grp(g):
                def jstep(j, carry):
                    a0, a1, a2, a3, cnt = carry
                    tokv = tok_d[b, j, pl.ds(g * 16, 16)]
                    cnt = cnt + jnp.where(tokv == 0, 1, 0).astype(jnp.int32)
                    accs = (a0, a1, a2, a3)
                    outs = []
                    for c in range(_W):
                        v = plsc.load_gather(
                            tab4, [jnp.full((16,), c, jnp.int32), tokv])
                        outs.append(accs[c] + v)
                    return outs[0], outs[1], outs[2], outs[3], cnt

                a0, a1, a2, a3, cnt = pl.loop(
                    0, _L, init_carry=(zf, zf, zf, zf, zi), unroll=5)(jstep)

                nf = cnt.astype(jnp.float32)
                inv = 1.0 / jnp.maximum(jnp.float32(_L) - nf, 1.0)
                accs = (a0, a1, a2, a3)
                for c in range(_W):
                    poolT[b, c, pl.ds(g * 16, 16)] = \
                        (accs[c] - nf * t0c[c]) * inv

            for c in range(_W):
                pltpu.async_copy(
                    poolT.at[b, c],
                    sh_pool.at[cbase + c, pl.ds(ch * _TCH, _TCH)], psems[b])

            @pl.when(pp < _NTCH // 2 - 1)
            def _pref():
                pltpu.async_copy(
                    tok.at[:, pl.ds(rbase_sc + (ch + 2) * _TCH, _TCH)],
                    tok_d.at[b], tsems[b])

    for b in range(2):
        for c in range(_W):
            pltpu.make_async_copy(
                poolT.at[b, c], sh_pool.at[c, pl.ds(0, _TCH)],
                psems[b]).wait()
    plsc.subcore_barrier()

    base0 = rbase_sc + sid * _RPW

    @pl.loop(0, _RPW // _PCH)
    def _piece(p):
        base = base0 + p * _PCH
        lrow = sid * _RPW + p * _PCH

        pltpu.sync_copy(lat_i.at[pl.ds(base, _PCH)], lli.at[0])
        pltpu.sync_copy(lon_i.at[pl.ds(base, _PCH)], lli.at[1])
        cp_lat = pltpu.async_copy(ltab.at[lli.at[0]], latb, sem_aux)
        cp_lon = pltpu.async_copy(ltab.at[lli.at[1]], lonb, sem_aux)

        pltpu.sync_copy(sh_pool.at[:, pl.ds(lrow, _PCH)], shT)

        @pl.loop(0, _PCH)
        def _tr(r):
            rsp = jnp.full((16,), r, jnp.int32)
            for k in range(_NSL):
                outp[r, pl.ds(k * 16, 16)] = plsc.load_gather(
                    shT, [lanes + k * 16, rsp])

        cp_lat.wait()
        cp_lon.wait()
        o1 = pltpu.async_copy(outp, out.at[pl.ds(base, _PCH), pl.ds(0, _D)],
                              sem_out)
        o2 = pltpu.async_copy(latb, out.at[pl.ds(base, _PCH), pl.ds(_D, _D)],
                              sem_out)
        o3 = pltpu.async_copy(lonb,
                              out.at[pl.ds(base, _PCH), pl.ds(2 * _D, _D)],
                              sem_out)
        o1.wait()
        o2.wait()
        o3.wait()


@jax.jit
def _run(tok, lat_i, lon_i, qtabT, t0, ltab):
    mesh = plsc.VectorSubcoreMesh(core_axis_name="c", subcore_axis_name="s")
    return pl.kernel(
        _sc_body,
        out_type=jax.ShapeDtypeStruct((_B, 3 * _D), jnp.float32),
        mesh=mesh,
        scratch_types=[
            pltpu.VMEM((_W, _VQ), jnp.float32),
            pltpu.VMEM((2, _L, _TCH), jnp.int32),
            pltpu.VMEM((2, _W, _TCH), jnp.float32),
            pltpu.VMEM((_D, _PCH), jnp.float32),
            pltpu.VMEM((_PCH, _D), jnp.float32),
            pltpu.VMEM((_PCH, _D), jnp.float32),
            pltpu.VMEM((_PCH, _D), jnp.float32),
            pltpu.VMEM((2, _PCH), jnp.int32),
            pltpu.VMEM((_D,), jnp.float32),
            pltpu.VMEM_SHARED((_D, _BSC), jnp.float32),
            pltpu.SemaphoreType.DMA,
            pltpu.SemaphoreType.DMA,
            pltpu.SemaphoreType.DMA,
            pltpu.SemaphoreType.DMA,
            pltpu.SemaphoreType.DMA,
            pltpu.SemaphoreType.DMA,
        ],
        compiler_params=pltpu.CompilerParams(use_tc_tiling_on_sc=False,
                                             needs_layout_passes=False),
        name="query_model_sc",
    )(tok, lat_i, lon_i, qtabT, t0, ltab)


def kernel(query_tokens, wh_latitude, wh_longitude, query_table, lonlat_table):
    tok = query_tokens.astype(jnp.int32).T
    lat_i = wh_latitude.astype(jnp.int32)
    lon_i = wh_longitude.astype(jnp.int32)
    qtab = query_table.astype(jnp.float32)
    return _run(tok, lat_i, lon_i, qtab.T, qtab[0],
                lonlat_table.astype(jnp.float32))


# speedup vs baseline: 1.1922x; 1.0074x over previous
import jax
import jax.numpy as jnp
from jax import lax
from jax.experimental import pallas as pl
from jax.experimental.pallas import tpu as pltpu
from jax.experimental.pallas import tpu_sc as plsc

_B = 16384
_L = 50
_D = 64
_VQ = 10000
_NC = 2
_NS = 16
_W = _D // _NS
_BSC = _B // _NC
_TCH = 128
_NTCH = _BSC // _TCH
_RPW = _B // (_NC * _NS)
_PCH = 128
_NSL = _D // 16


def _sc_body(tok, lat_i, lon_i, qtabT, t0, ltab, out,
             tab4, tok_d, poolT, shT, outp, latb, lonb, lli, t0v, sh_pool,
             sem_t0, sem_t1, sem_p0, sem_p1, sem_aux, sem_out):
    sid = lax.axis_index("s")
    cid = lax.axis_index("c")
    rbase_sc = cid * _BSC
    cbase = sid * _W

    pltpu.sync_copy(qtabT.at[pl.ds(cbase, _W), :], tab4)
    pltpu.sync_copy(t0, t0v)

    lanes = lax.broadcasted_iota(jnp.int32, (16,), 0)
    zf = jnp.zeros((16,), jnp.float32)
    zi = jnp.zeros((16,), jnp.int32)
    tsems = (sem_t0, sem_t1)
    psems = (sem_p0, sem_p1)

    for b in range(2):
        pltpu.async_copy(tok.at[:, pl.ds(rbase_sc + b * _TCH, _TCH)],
                         tok_d.at[b], tsems[b])

    t0c = [plsc.load_gather(t0v, [jnp.full((16,), cbase + c, jnp.int32)])
           for c in range(_W)]

    @pl.loop(0, _NTCH // 2)
    def _pair(pp):
        for b in range(2):
            ch = pp * 2 + b
            pltpu.make_async_copy(tok.at[:, pl.ds(0, _TCH)], tok_d.at[b],
                                  tsems[b]).wait()

            @pl.when(pp > 0)
            def _drain():
                for c in range(_W):
                    pltpu.make_async_copy(
                        poolT.at[b, c], sh_pool.at[c, pl.ds(0, _TCH)],
                        psems[b]).wait()

            @pl.loop(0, _TCH // 16)
            def _grp(g):
                def jstep(j, carry):
                    a0, a1, a2, a3, cnt = carry
                    tokv = tok_d[b, j, pl.ds(g * 16, 16)]
                    cnt = cnt + jnp.where(tokv == 0, 1, 0).astype(jnp.int32)
                    accs = (a0, a1, a2, a3)
                    outs = []
                    for c in range(_W):
                        v = plsc.load_gather(
                            tab4, [jnp.full((16,), c, jnp.int32), tokv])
                        outs.append(accs[c] + v)
                    return outs[0], outs[1], outs[2], outs[3], cnt

                a0, a1, a2, a3, cnt = pl.loop(
                    0, _L, init_carry=(zf, zf, zf, zf, zi), unroll=10)(jstep)

                nf = cnt.astype(jnp.float32)
                inv = 1.0 / jnp.maximum(jnp.float32(_L) - nf, 1.0)
                accs = (a0, a1, a2, a3)
                for c in range(_W):
                    poolT[b, c, pl.ds(g * 16, 16)] = \
                        (accs[c] - nf * t0c[c]) * inv

            for c in range(_W):
                pltpu.async_copy(
                    poolT.at[b, c],
                    sh_pool.at[cbase + c, pl.ds(ch * _TCH, _TCH)], psems[b])

            @pl.when(pp < _NTCH // 2 - 1)
            def _pref():
                pltpu.async_copy(
                    tok.at[:, pl.ds(rbase_sc + (ch + 2) * _TCH, _TCH)],
                    tok_d.at[b], tsems[b])

    for b in range(2):
        for c in range(_W):
            pltpu.make_async_copy(
                poolT.at[b, c], sh_pool.at[c, pl.ds(0, _TCH)],
                psems[b]).wait()
    plsc.subcore_barrier()

    base0 = rbase_sc + sid * _RPW

    @pl.loop(0, _RPW // _PCH)
    def _piece(p):
        base = base0 + p * _PCH
        lrow = sid * _RPW + p * _PCH

        pltpu.sync_copy(lat_i.at[pl.ds(base, _PCH)], lli.at[0])
        pltpu.sync_copy(lon_i.at[pl.ds(base, _PCH)], lli.at[1])
        cp_lat = pltpu.async_copy(ltab.at[lli.at[0]], latb, sem_aux)
        cp_lon = pltpu.async_copy(ltab.at[lli.at[1]], lonb, sem_aux)

        pltpu.sync_copy(sh_pool.at[:, pl.ds(lrow, _PCH)], shT)

        @pl.loop(0, _PCH)
        def _tr(r):
            rsp = jnp.full((16,), r, jnp.int32)
            for k in range(_NSL):
                outp[r, pl.ds(k * 16, 16)] = plsc.load_gather(
                    shT, [lanes + k * 16, rsp])

        cp_lat.wait()
        cp_lon.wait()
        o1 = pltpu.async_copy(outp, out.at[pl.ds(base, _PCH), pl.ds(0, _D)],
                              sem_out)
        o2 = pltpu.async_copy(latb, out.at[pl.ds(base, _PCH), pl.ds(_D, _D)],
                              sem_out)
        o3 = pltpu.async_copy(lonb,
                              out.at[pl.ds(base, _PCH), pl.ds(2 * _D, _D)],
                              sem_out)
        o1.wait()
        o2.wait()
        o3.wait()


@jax.jit
def _run(tok, lat_i, lon_i, qtabT, t0, ltab):
    mesh = plsc.VectorSubcoreMesh(core_axis_name="c", subcore_axis_name="s")
    return pl.kernel(
        _sc_body,
        out_type=jax.ShapeDtypeStruct((_B, 3 * _D), jnp.float32),
        mesh=mesh,
        scratch_types=[
            pltpu.VMEM((_W, _VQ), jnp.float32),
            pltpu.VMEM((2, _L, _TCH), jnp.int32),
            pltpu.VMEM((2, _W, _TCH), jnp.float32),
            pltpu.VMEM((_D, _PCH), jnp.float32),
            pltpu.VMEM((_PCH, _D), jnp.float32),
            pltpu.VMEM((_PCH, _D), jnp.float32),
            pltpu.VMEM((_PCH, _D), jnp.float32),
            pltpu.VMEM((2, _PCH), jnp.int32),
            pltpu.VMEM((_D,), jnp.float32),
            pltpu.VMEM_SHARED((_D, _BSC), jnp.float32),
            pltpu.SemaphoreType.DMA,
            pltpu.SemaphoreType.DMA,
            pltpu.SemaphoreType.DMA,
            pltpu.SemaphoreType.DMA,
            pltpu.SemaphoreType.DMA,
            pltpu.SemaphoreType.DMA,
        ],
        compiler_params=pltpu.CompilerParams(use_tc_tiling_on_sc=False,
                                             needs_layout_passes=False),
        name="query_model_sc",
    )(tok, lat_i, lon_i, qtabT, t0, ltab)


def kernel(query_tokens, wh_latitude, wh_longitude, query_table, lonlat_table):
    tok = query_tokens.astype(jnp.int32).T
    lat_i = wh_latitude.astype(jnp.int32)
    lon_i = wh_longitude.astype(jnp.int32)
    qtab = query_table.astype(jnp.float32)
    return _run(tok, lat_i, lon_i, qtab.T, qtab[0],
                lonlat_table.astype(jnp.float32))
